# baseline (speedup 1.0000x reference)
import jax
import jax.numpy as jnp
from jax.experimental import pallas as pl
from jax.experimental.pallas import tpu as pltpu

_BATCH = 16384
_STATE_DIM = 256
_HIDDEN = 1024
_NUM_ACTIONS = 257
_ROWS = 512


def _fused_body(s_ref, w1_ref, b1_ref, w2_ref, b2_ref, o_ref,
                w1bf_ref, w2bf_ref):
    @pl.when(pl.program_id(0) == 0)
    def _cast_weights():
        w1bf_ref[...] = w1_ref[...].astype(jnp.bfloat16)
        w2bf_ref[...] = w2_ref[...].astype(jnp.bfloat16)

    s = s_ref[...]
    h = jnp.dot(s.astype(jnp.bfloat16), w1bf_ref[...],
                preferred_element_type=jnp.float32)
    h = jnp.maximum(h + b1_ref[...], 0.0).astype(jnp.bfloat16)
    logits = jnp.dot(h, w2bf_ref[...], preferred_element_type=jnp.float32)
    logits = logits + b2_ref[...]
    mx = jnp.max(logits, axis=1, keepdims=True)
    e = jnp.exp(logits - mx)
    cont = jnp.where(s < 2.0, e[:, :_STATE_DIM], 0.0)
    term = e[:, _STATE_DIM:]
    denom = jnp.sum(cont, axis=1, keepdims=True) + term
    o_ref[:, :_STATE_DIM] = cont / denom
    o_ref[:, _STATE_DIM:] = term / denom


def kernel(states, W1, b1, W2, b2):
    grid = (_BATCH // _ROWS,)
    return pl.pallas_call(
        _fused_body,
        grid=grid,
        in_specs=[
            pl.BlockSpec((_ROWS, _STATE_DIM), lambda i: (i, 0)),
            pl.BlockSpec((_STATE_DIM, _HIDDEN), lambda i: (0, 0)),
            pl.BlockSpec((1, _HIDDEN), lambda i: (0, 0)),
            pl.BlockSpec((_HIDDEN, _NUM_ACTIONS), lambda i: (0, 0)),
            pl.BlockSpec((1, _NUM_ACTIONS), lambda i: (0, 0)),
        ],
        out_specs=pl.BlockSpec((_ROWS, _NUM_ACTIONS), lambda i: (i, 0)),
        out_shape=jax.ShapeDtypeStruct((_BATCH, _NUM_ACTIONS), jnp.float32),
        scratch_shapes=[
            pltpu.VMEM((_STATE_DIM, _HIDDEN), jnp.bfloat16),
            pltpu.VMEM((_HIDDEN, _NUM_ACTIONS), jnp.bfloat16),
        ],
        compiler_params=pltpu.CompilerParams(
            dimension_semantics=("arbitrary",),
        ),
    )(states, W1, b1.reshape(1, _HIDDEN), W2, b2.reshape(1, _NUM_ACTIONS))


# speedup vs baseline: 1.1261x; 1.1261x over previous
import jax
import jax.numpy as jnp
from jax.experimental import pallas as pl
from jax.experimental.pallas import tpu as pltpu

_BATCH = 16384
_STATE_DIM = 256
_HIDDEN = 1024
_NUM_ACTIONS = 257
_PAD = 384
_ROWS = 512


def _fused_body(s_ref, w1_ref, b1_ref, w2_ref, b2_ref, o_ref,
                w1bf_ref, w2bf_ref, b2p_ref):
    @pl.when(pl.program_id(0) == 0)
    def _prep_weights():
        w1bf_ref[...] = w1_ref[...].astype(jnp.bfloat16)
        w2bf_ref[...] = jnp.zeros((_HIDDEN, _PAD), jnp.bfloat16)
        w2bf_ref[:, :_NUM_ACTIONS] = w2_ref[...].astype(jnp.bfloat16)
        b2p_ref[...] = jnp.full((1, _PAD), -1e9, jnp.float32)
        b2p_ref[:, :_NUM_ACTIONS] = b2_ref[...]

    s = s_ref[...]
    h = jnp.dot(s.astype(jnp.bfloat16), w1bf_ref[...],
                preferred_element_type=jnp.float32)
    h = jnp.maximum(h + b1_ref[...], 0.0).astype(jnp.bfloat16)
    logits = jnp.dot(h, w2bf_ref[...], preferred_element_type=jnp.float32)
    logits = logits + b2p_ref[...]
    mx = jnp.max(logits, axis=1, keepdims=True)
    e = jnp.exp(logits - mx)
    cont = (s < 2.0).astype(jnp.float32)
    col = jax.lax.broadcasted_iota(jnp.int32, (s.shape[0], 128), 1)
    tail = (col == 0).astype(jnp.float32)
    mask = jnp.concatenate([cont, tail], axis=1)
    me = e * mask
    out = me / jnp.sum(me, axis=1, keepdims=True)
    o_ref[...] = out[:, :_NUM_ACTIONS]


def kernel(states, W1, b1, W2, b2):
    grid = (_BATCH // _ROWS,)
    return pl.pallas_call(
        _fused_body,
        grid=grid,
        in_specs=[
            pl.BlockSpec((_ROWS, _STATE_DIM), lambda i: (i, 0)),
            pl.BlockSpec((_STATE_DIM, _HIDDEN), lambda i: (0, 0)),
            pl.BlockSpec((1, _HIDDEN), lambda i: (0, 0)),
            pl.BlockSpec((_HIDDEN, _NUM_ACTIONS), lambda i: (0, 0)),
            pl.BlockSpec((1, _NUM_ACTIONS), lambda i: (0, 0)),
        ],
        out_specs=pl.BlockSpec((_ROWS, _NUM_ACTIONS), lambda i: (i, 0)),
        out_shape=jax.ShapeDtypeStruct((_BATCH, _NUM_ACTIONS), jnp.float32),
        scratch_shapes=[
            pltpu.VMEM((_STATE_DIM, _HIDDEN), jnp.bfloat16),
            pltpu.VMEM((_HIDDEN, _PAD), jnp.bfloat16),
            pltpu.VMEM((1, _PAD), jnp.float32),
        ],
        compiler_params=pltpu.CompilerParams(
            dimension_semantics=("arbitrary",),
        ),
    )(states, W1, b1.reshape(1, _HIDDEN), W2, b2.reshape(1, _NUM_ACTIONS))
